# baseline (device time: 102024 ns/iter reference)
import jax
import jax.numpy as jnp
from jax import lax
from jax.experimental import pallas as pl
from jax.experimental.pallas import tpu as pltpu

N_DEV = 32
N_HOP = N_DEV // 2

_TABLES = None


def _ring_tables():
    global _TABLES
    if _TABLES is None:
        import distributed_mesh_v7x as dm

        mesh = dm.get_mesh("i", world_size=N_DEV)
        coords = [tuple(d.coords) for d in mesh.devices.flat]
        pos_of = {c: p for p, c in enumerate(coords)}
        snake = [(0, 0), (1, 0), (2, 0), (3, 0),
                 (3, 1), (2, 1), (1, 1), (0, 1),
                 (0, 2), (1, 2), (2, 2), (3, 2),
                 (3, 3), (2, 3), (1, 3), (0, 3)]
        ham = ([(0, y, z) for (y, z) in snake]
               + [(1, y, z) for (y, z) in reversed(snake)])
        hp = [pos_of[c] for c in ham]
        nxt = [0] * N_DEV
        prv = [0] * N_DEV
        hpos = [0] * N_DEV
        for idx, p in enumerate(hp):
            nxt[p] = hp[(idx + 1) % N_DEV]
            prv[p] = hp[(idx - 1) % N_DEV]
            hpos[p] = idx
        _TABLES = (hp, nxt, prv, hpos)
    return _TABLES


def kernel(x, w_mat):
    m_per, k = x.shape
    _, n_per = w_mat.shape
    m_glob = N_DEV * m_per
    half = m_per // 2

    hp, nxt, prv, hpos = _ring_tables()
    my = lax.axis_index("i")
    scalars = jnp.stack([
        jnp.take(jnp.array(nxt, jnp.int32), my),
        jnp.take(jnp.array(prv, jnp.int32), my),
        jnp.take(jnp.array(hpos, jnp.int32), my),
    ])
    hp_arr = jnp.array(hp, jnp.int32)

    def body(scal_ref, hp_ref, x_ref, w_ref, out_ref, buf_cw, buf_ccw,
             send_cw, recv_cw, send_ccw, recv_ccw, dummy_sem):
        right = scal_ref[0]
        left = scal_ref[1]
        myh = scal_ref[2]

        def origin(off):
            return hp_ref[lax.rem(myh + off + N_DEV, N_DEV)]

        barrier_sem = pltpu.get_barrier_semaphore()
        for nbr in (left, right):
            pl.semaphore_signal(
                barrier_sem, inc=1,
                device_id=(nbr,), device_id_type=pl.DeviceIdType.MESH,
            )
        pl.semaphore_wait(barrier_sem, 2)

        rows = lambda i: pl.ds(i * half, half)
        sends = []

        def start_send(src, dst, send_sem, recv_sem, target):
            r = pltpu.make_async_remote_copy(
                src_ref=src, dst_ref=dst,
                send_sem=send_sem, recv_sem=recv_sem,
                device_id=(target,), device_id_type=pl.DeviceIdType.MESH,
            )
            r.start()
            sends.append(r)

        def wait_recv(dst, recv_sem):
            pltpu.make_async_remote_copy(
                src_ref=dst, dst_ref=dst,
                send_sem=dummy_sem.at[0], recv_sem=recv_sem,
                device_id=(left,), device_id_type=pl.DeviceIdType.MESH,
            ).wait_recv()

        def relu_dot_store(chunk, orig):
            acc = jnp.dot(chunk, w_ref[:, :],
                          preferred_element_type=jnp.float32)
            out_ref[pl.ds(orig * m_per, m_per), :] = jnp.maximum(acc, 0.0)

        def band_dot(buf, sgn, lo, hi):
            band = jnp.reshape(buf[lo:hi, :, :], ((hi - lo) * m_per, k))
            acc = jnp.dot(band, w_ref[:, :], preferred_element_type=jnp.float32)
            acc = jnp.maximum(acc, 0.0)
            for s in range(lo, hi):
                out_ref[pl.ds(origin(sgn * s) * m_per, m_per), :] = (
                    acc[(s - lo) * m_per:(s - lo + 1) * m_per, :])

        for i in (0, 1):
            start_send(x_ref.at[rows(i), :], buf_cw.at[1, rows(i), :],
                       send_cw.at[0 * 2 + i], recv_cw.at[1 * 2 + i], right)
        for i in (0, 1):
            start_send(x_ref.at[rows(i), :], buf_ccw.at[1, rows(i), :],
                       send_ccw.at[0 * 2 + i], recv_ccw.at[1 * 2 + i], left)

        relu_dot_store(x_ref[:, :], origin(0))

        for d in range(1, N_HOP):
            for i in (0, 1):
                wait_recv(buf_cw.at[d, rows(i), :], recv_cw.at[d * 2 + i])
                if d <= N_HOP - 2:
                    start_send(buf_cw.at[d, rows(i), :],
                               buf_cw.at[d + 1, rows(i), :],
                               send_cw.at[d * 2 + i],
                               recv_cw.at[(d + 1) * 2 + i], right)
                elif i == 0:
                    start_send(buf_cw.at[d, rows(0), :],
                               buf_cw.at[d + 1, rows(0), :],
                               send_cw.at[d * 2 + 0],
                               recv_cw.at[(d + 1) * 2 + 0], right)
            for i in (0, 1):
                wait_recv(buf_ccw.at[d, rows(i), :], recv_ccw.at[d * 2 + i])
                if d <= N_HOP - 2:
                    start_send(buf_ccw.at[d, rows(i), :],
                               buf_ccw.at[d + 1, rows(i), :],
                               send_ccw.at[d * 2 + i],
                               recv_ccw.at[(d + 1) * 2 + i], left)
                elif i == 1:
                    start_send(buf_ccw.at[d, rows(1), :],
                               buf_ccw.at[d + 1, rows(1), :],
                               send_ccw.at[d * 2 + 1],
                               recv_ccw.at[(d + 1) * 2 + 1], left)
            if d == 8:
                band_dot(buf_cw, -1, 1, 9)
                band_dot(buf_ccw, +1, 1, 9)

        band_dot(buf_cw, -1, 9, N_HOP)
        band_dot(buf_ccw, +1, 9, N_HOP)

        o_anti = origin(N_HOP)
        wait_recv(buf_cw.at[N_HOP, rows(0), :], recv_cw.at[N_HOP * 2 + 0])
        wait_recv(buf_ccw.at[N_HOP, rows(1), :], recv_ccw.at[N_HOP * 2 + 1])
        for buf, i in ((buf_cw, 0), (buf_ccw, 1)):
            acc = jnp.dot(buf[N_HOP, i * half:(i + 1) * half, :], w_ref[:, :],
                          preferred_element_type=jnp.float32)
            out_ref[pl.ds(o_anti * m_per + i * half, half), :] = (
                jnp.maximum(acc, 0.0))

        for r in sends:
            r.wait_send()

    return pl.pallas_call(
        body,
        out_shape=jax.ShapeDtypeStruct((m_glob, n_per), jnp.float32),
        in_specs=[
            pl.BlockSpec(memory_space=pltpu.SMEM),
            pl.BlockSpec(memory_space=pltpu.SMEM),
            pl.BlockSpec(memory_space=pltpu.VMEM),
            pl.BlockSpec(memory_space=pltpu.VMEM),
        ],
        out_specs=pl.BlockSpec(memory_space=pltpu.VMEM),
        scratch_shapes=[
            pltpu.VMEM((N_HOP + 1, m_per, k), jnp.float32),
            pltpu.VMEM((N_HOP + 1, m_per, k), jnp.float32),
            pltpu.SemaphoreType.DMA((2 * N_HOP,)),
            pltpu.SemaphoreType.DMA((2 * (N_HOP + 1),)),
            pltpu.SemaphoreType.DMA((2 * N_HOP,)),
            pltpu.SemaphoreType.DMA((2 * (N_HOP + 1),)),
            pltpu.SemaphoreType.DMA((1,)),
        ],
        compiler_params=pltpu.CompilerParams(collective_id=0),
    )(scalars, hp_arr, x, w_mat)


# device time: 100907 ns/iter; 1.0111x vs baseline; 1.0111x over previous
import jax
import jax.numpy as jnp
from jax import lax
from jax.experimental import pallas as pl
from jax.experimental.pallas import tpu as pltpu

N_DEV = 32
N_HOP = N_DEV // 2

_TABLES = None


def _ring_tables():
    global _TABLES
    if _TABLES is None:
        import distributed_mesh_v7x as dm

        mesh = dm.get_mesh("i", world_size=N_DEV)
        coords = [tuple(d.coords) for d in mesh.devices.flat]
        pos_of = {c: p for p, c in enumerate(coords)}
        snake = [(0, 0), (1, 0), (2, 0), (3, 0),
                 (3, 1), (2, 1), (1, 1), (0, 1),
                 (0, 2), (1, 2), (2, 2), (3, 2),
                 (3, 3), (2, 3), (1, 3), (0, 3)]
        ham = ([(0, y, z) for (y, z) in snake]
               + [(1, y, z) for (y, z) in reversed(snake)])
        hp = [pos_of[c] for c in ham]
        nxt = [0] * N_DEV
        prv = [0] * N_DEV
        hpos = [0] * N_DEV
        for idx, p in enumerate(hp):
            nxt[p] = hp[(idx + 1) % N_DEV]
            prv[p] = hp[(idx - 1) % N_DEV]
            hpos[p] = idx
        _TABLES = (hp, nxt, prv, hpos)
    return _TABLES


def kernel(x, w_mat):
    m_per, k = x.shape
    _, n_per = w_mat.shape
    m_glob = N_DEV * m_per
    half = m_per // 2

    hp, nxt, prv, hpos = _ring_tables()
    my = lax.axis_index("i")
    scalars = jnp.stack([
        jnp.take(jnp.array(nxt, jnp.int32), my),
        jnp.take(jnp.array(prv, jnp.int32), my),
        jnp.take(jnp.array(hpos, jnp.int32), my),
    ])
    hp_arr = jnp.array(hp, jnp.int32)

    def body(scal_ref, hp_ref, x_ref, w_ref, out_ref, buf_cw, buf_ccw,
             send_cw, recv_cw, send_ccw, recv_ccw, dummy_sem):
        right = scal_ref[0]
        left = scal_ref[1]
        myh = scal_ref[2]

        def origin(off):
            return hp_ref[lax.rem(myh + off + N_DEV, N_DEV)]

        barrier_sem = pltpu.get_barrier_semaphore()
        for nbr in (left, right):
            pl.semaphore_signal(
                barrier_sem, inc=1,
                device_id=(nbr,), device_id_type=pl.DeviceIdType.MESH,
            )
        pl.semaphore_wait(barrier_sem, 2)

        rows = lambda i: pl.ds(i * half, half)
        sends = []

        def start_send(src, dst, send_sem, recv_sem, target):
            r = pltpu.make_async_remote_copy(
                src_ref=src, dst_ref=dst,
                send_sem=send_sem, recv_sem=recv_sem,
                device_id=(target,), device_id_type=pl.DeviceIdType.MESH,
            )
            r.start()
            sends.append(r)

        def wait_recv(dst, recv_sem):
            pltpu.make_async_remote_copy(
                src_ref=dst, dst_ref=dst,
                send_sem=dummy_sem.at[0], recv_sem=recv_sem,
                device_id=(left,), device_id_type=pl.DeviceIdType.MESH,
            ).wait_recv()

        def relu_dot_store(chunk, orig):
            acc = jnp.dot(chunk, w_ref[:, :],
                          preferred_element_type=jnp.float32)
            out_ref[pl.ds(orig * m_per, m_per), :] = jnp.maximum(acc, 0.0)

        for i in (0, 1):
            start_send(x_ref.at[rows(i), :], buf_cw.at[1, rows(i), :],
                       send_cw.at[0 * 2 + i], recv_cw.at[1 * 2 + i], right)
        for i in (0, 1):
            start_send(x_ref.at[rows(i), :], buf_ccw.at[1, rows(i), :],
                       send_ccw.at[0 * 2 + i], recv_ccw.at[1 * 2 + i], left)

        relu_dot_store(x_ref[:, :], origin(0))

        for d in range(1, N_HOP):
            for i in (0, 1):
                wait_recv(buf_cw.at[d, rows(i), :], recv_cw.at[d * 2 + i])
                if d <= N_HOP - 2:
                    start_send(buf_cw.at[d, rows(i), :],
                               buf_cw.at[d + 1, rows(i), :],
                               send_cw.at[d * 2 + i],
                               recv_cw.at[(d + 1) * 2 + i], right)
                elif i == 0:
                    start_send(buf_cw.at[d, rows(0), :],
                               buf_cw.at[d + 1, rows(0), :],
                               send_cw.at[d * 2 + 0],
                               recv_cw.at[(d + 1) * 2 + 0], right)
            for i in (0, 1):
                wait_recv(buf_ccw.at[d, rows(i), :], recv_ccw.at[d * 2 + i])
                if d <= N_HOP - 2:
                    start_send(buf_ccw.at[d, rows(i), :],
                               buf_ccw.at[d + 1, rows(i), :],
                               send_ccw.at[d * 2 + i],
                               recv_ccw.at[(d + 1) * 2 + i], left)
                elif i == 1:
                    start_send(buf_ccw.at[d, rows(1), :],
                               buf_ccw.at[d + 1, rows(1), :],
                               send_ccw.at[d * 2 + 1],
                               recv_ccw.at[(d + 1) * 2 + 1], left)

        for buf, sgn in ((buf_cw, -1), (buf_ccw, +1)):
            band = jnp.reshape(buf[1:N_HOP, :, :], ((N_HOP - 1) * m_per, k))
            acc = jnp.dot(band, w_ref[:, :], preferred_element_type=jnp.float32)
            acc = jnp.maximum(acc, 0.0)
            for c in range(N_HOP - 1):
                orig = origin(sgn * (c + 1))
                out_ref[pl.ds(orig * m_per, m_per), :] = (
                    acc[c * m_per:(c + 1) * m_per, :])

        o_anti = origin(N_HOP)
        wait_recv(buf_cw.at[N_HOP, rows(0), :], recv_cw.at[N_HOP * 2 + 0])
        wait_recv(buf_ccw.at[N_HOP, rows(1), :], recv_ccw.at[N_HOP * 2 + 1])
        for buf, i in ((buf_cw, 0), (buf_ccw, 1)):
            acc = jnp.dot(buf[N_HOP, i * half:(i + 1) * half, :], w_ref[:, :],
                          preferred_element_type=jnp.float32)
            out_ref[pl.ds(o_anti * m_per + i * half, half), :] = (
                jnp.maximum(acc, 0.0))

        for r in sends:
            r.wait_send()

    return pl.pallas_call(
        body,
        out_shape=jax.ShapeDtypeStruct((m_glob, n_per), jnp.float32),
        in_specs=[
            pl.BlockSpec(memory_space=pltpu.SMEM),
            pl.BlockSpec(memory_space=pltpu.SMEM),
            pl.BlockSpec(memory_space=pltpu.VMEM),
            pl.BlockSpec(memory_space=pltpu.VMEM),
        ],
        out_specs=pl.BlockSpec(memory_space=pltpu.VMEM),
        scratch_shapes=[
            pltpu.VMEM((N_HOP + 1, m_per, k), jnp.float32),
            pltpu.VMEM((N_HOP + 1, m_per, k), jnp.float32),
            pltpu.SemaphoreType.DMA((2 * N_HOP,)),
            pltpu.SemaphoreType.DMA((2 * (N_HOP + 1),)),
            pltpu.SemaphoreType.DMA((2 * N_HOP,)),
            pltpu.SemaphoreType.DMA((2 * (N_HOP + 1),)),
            pltpu.SemaphoreType.DMA((1,)),
        ],
        compiler_params=pltpu.CompilerParams(collective_id=0),
    )(scalars, hp_arr, x, w_mat)
